# Initial kernel scaffold; baseline (speedup 1.0000x reference)
import jax
import jax.numpy as jnp
from jax.experimental import pallas as pl


def kernel(h, edge_attr, We, be, ge, bte, Wm1, bm1, Wm2, bm2, gmsg, bmsg, Wf1, bf1, Wf2, bf2, gffn, bffn, edge_index):
    raise NotImplementedError("write your pallas kernel here")


# speedup vs baseline: 5.1427x; 5.1427x over previous
import functools

import jax
import jax.numpy as jnp
from jax import lax
from jax.experimental import pallas as pl
from jax.experimental.pallas import tpu as pltpu
from jax.experimental.pallas import tpu_sc as plsc

F32 = jnp.float32

_GC1 = 2.0 * 0.7978845608028654
_GC2 = _GC1 * 0.044715


def _gelu_exact(x):
    return 0.5 * x * (1.0 + lax.erf(x * 0.7071067811865476))


def _layer_norm(x, g, b, eps=1e-5):
    mu = jnp.mean(x, axis=-1, keepdims=True)
    var = jnp.mean((x - mu) ** 2, axis=-1, keepdims=True)
    return (x - mu) / jnp.sqrt(var + eps) * g + b


def _hw_body(h_ref, w_ref, out_ref):
    x = jnp.dot(h_ref[...], w_ref[...], preferred_element_type=F32)
    half = x.shape[-1] // 2
    out_ref[0] = x[:, :half]
    out_ref[1] = x[:, half:]


def _hw_call(h2d, w_top):
    BN, H = h2d.shape
    R = 1000
    return pl.pallas_call(
        _hw_body,
        grid=(BN // R,),
        in_specs=[
            pl.BlockSpec((R, H), lambda i: (i, 0)),
            pl.BlockSpec((H, H), lambda i: (0, 0)),
        ],
        out_specs=pl.BlockSpec((2, R, H // 2), lambda i: (0, i, 0)),
        out_shape=jax.ShapeDtypeStruct((2, BN, H // 2), F32),
    )(h2d, w_top)


def _enc_body(ea_ref, we_ref, be_ref, ge_ref, bte_ref, wb_ref, bm1_ref, out_ref):
    t = jnp.dot(ea_ref[...], we_ref[...], preferred_element_type=F32) + be_ref[...]
    t = _layer_norm(t, ge_ref[...], bte_ref[...])
    t = _gelu_exact(t)
    x = jnp.dot(t, wb_ref[...], preferred_element_type=F32) + bm1_ref[...]
    half = x.shape[-1] // 2
    out_ref[0] = x[:, :half]
    out_ref[1] = x[:, half:]


def _enc_call(edge_attr, We, be, ge, bte, w_bot, bm1):
    E, A = edge_attr.shape
    EH, H = w_bot.shape
    R = 1000
    be2, ge2, bte2, bm12 = (v.reshape(1, -1) for v in (be, ge, bte, bm1))
    return pl.pallas_call(
        _enc_body,
        grid=(E // R,),
        in_specs=[
            pl.BlockSpec((R, A), lambda i: (i, 0)),
            pl.BlockSpec((A, EH), lambda i: (0, 0)),
            pl.BlockSpec((1, EH), lambda i: (0, 0)),
            pl.BlockSpec((1, EH), lambda i: (0, 0)),
            pl.BlockSpec((1, EH), lambda i: (0, 0)),
            pl.BlockSpec((EH, H), lambda i: (0, 0)),
            pl.BlockSpec((1, H), lambda i: (0, 0)),
        ],
        out_specs=pl.BlockSpec((2, R, H // 2), lambda i: (0, i, 0)),
        out_shape=jax.ShapeDtypeStruct((2, E, H // 2), F32),
    )(edge_attr, We, be2, ge2, bte2, w_bot, bm12)


_NC, _NS = 2, 16
_K = 80
_ZR = 128


def _sc_agg(hw2, encw2, src, dst, B, N, E, HALF):
    EPT = E // _NS
    NCH = EPT // _K
    NP = -(-N // (_NS * _ZR)) * (_NS * _ZR)
    NPT = NP // _NS

    NTC = NPT // _K
    QC = NP // _K

    def body(hw_hbm, encw_hbm, src_hbm, dst_hbm, agg_hbm,
             sidx, didx, gidx, encb, rows, agg_sh, sem):
        c = lax.axis_index("c")
        s = lax.axis_index("s")

        zero16 = jnp.zeros((16,), F32)
        one16 = jnp.ones((16,), F32)

        for b in range(B + 1):
            def zrow_body(r, _):
                for l in range(HALF // 16):
                    rows[r, pl.ds(l * 16, 16)] = zero16
                return 0
            lax.fori_loop(0, _K, zrow_body, 0)
            for t in range(NTC):
                pltpu.sync_copy(rows, agg_sh.at[pl.ds(s * NPT + t * _K, _K)])
            plsc.subcore_barrier()

            if b < B:
                off = (c * B + b) * N

                def chunk_body(j, _):
                    base = s * EPT + j * _K
                    pltpu.sync_copy(src_hbm.at[pl.ds(base, _K)], sidx)
                    pltpu.sync_copy(dst_hbm.at[pl.ds(base, _K)], didx)
                    pltpu.sync_copy(encw_hbm.at[c * (E // _K) + s * NCH + j], encb)
                    for v in range(_K // 16):
                        gvec = sidx[pl.ds(v * 16, 16)] + off
                        pltpu.async_copy(hw_hbm.at[gvec],
                                         rows.at[pl.ds(v * 16, 16)], sem).wait()

                    def row_body(r, _):
                        for l in range(HALF // 16):
                            x = rows[r, pl.ds(l * 16, 16)] + encb[r, pl.ds(l * 16, 16)]
                            xx = x * x
                            arg = x * (-_GC1 - _GC2 * xx)
                            rows[r, pl.ds(l * 16, 16)] = x / (1.0 + jnp.exp(arg))
                        return 0
                    lax.fori_loop(0, _K, row_body, 0)

                    for v in range(_K // 16):
                        dvec = didx[pl.ds(v * 16, 16)]
                        pltpu.sync_copy(rows.at[pl.ds(v * 16, 16)],
                                        agg_sh.at[dvec], add=True)
                    return 0
                lax.fori_loop(0, NCH, chunk_body, 0)
            else:
                def orow_body(r, _):
                    for l in range(HALF // 16):
                        rows[r, pl.ds(l * 16, 16)] = one16
                    return 0
                lax.fori_loop(0, _K, orow_body, 0)

                def deg_body(j, _):
                    base = s * EPT + j * _K
                    pltpu.sync_copy(dst_hbm.at[pl.ds(base, _K)], didx)
                    for v in range(_K // 16):
                        dvec = didx[pl.ds(v * 16, 16)]
                        pltpu.sync_copy(rows.at[pl.ds(v * 16, 16)],
                                        agg_sh.at[dvec], add=True)
                    return 0
                lax.fori_loop(0, NCH, deg_body, 0)
            plsc.subcore_barrier()

            for t in range(NTC):
                pltpu.sync_copy(agg_sh.at[pl.ds(s * NPT + t * _K, _K)], rows)
                pltpu.sync_copy(rows,
                                agg_hbm.at[(c * (B + 1) + b) * QC + s * NTC + t])

    mesh = plsc.VectorSubcoreMesh(core_axis_name="c", subcore_axis_name="s",
                                  num_cores=_NC, num_subcores=_NS)
    agg = pl.kernel(
        body,
        out_type=jax.ShapeDtypeStruct((_NC * (B + 1) * NP // _K, _K, HALF), F32),
        mesh=mesh,
        scratch_types=[
            pltpu.VMEM((_K,), jnp.int32),
            pltpu.VMEM((_K,), jnp.int32),
            pltpu.VMEM((_K,), jnp.int32),
            pltpu.VMEM((_K, HALF), F32),
            pltpu.VMEM((_K, HALF), F32),
            pltpu.VMEM_SHARED((NP, HALF), F32),
            pltpu.SemaphoreType.DMA,
        ],
    )(hw2, encw2.reshape(_NC * E // _K, _K, HALF), src, dst)
    return agg.reshape(_NC, B + 1, NP, HALF), NP


def _post_body(h_ref, agg_ref, deg_ref, wm2_ref, bm2_ref, gm_ref, bm_ref,
               wf1_ref, bf1_ref, wf2_ref, bf2_ref, gf_ref, bf_ref, out_ref):
    deg = deg_ref[0, 0][:, 0:1]
    inv = 1.0 / jnp.maximum(deg, 1.0)
    present = jnp.where(deg > 0.0, 1.0, 0.0)
    aggn = jnp.concatenate([agg_ref[0, 0], agg_ref[1, 0]], axis=-1) * inv
    aggregated = (jnp.dot(aggn, wm2_ref[...], preferred_element_type=F32)
                  + bm2_ref[...] * present)
    h1 = _layer_norm(h_ref[0] + aggregated, gm_ref[...], bm_ref[...])
    f = _gelu_exact(jnp.dot(h1, wf1_ref[...], preferred_element_type=F32)
                    + bf1_ref[...])
    ffn = jnp.dot(f, wf2_ref[...], preferred_element_type=F32) + bf2_ref[...]
    out_ref[0] = _layer_norm(h1 + ffn, gf_ref[...], bf_ref[...])


def _post_call(h, agg4, Wm2, bm2, gmsg, bmsg, Wf1, bf1, Wf2, bf2, gffn, bffn):
    B, N, H = h.shape
    RN = 1000
    v2 = (bm2, gmsg, bmsg, gffn, bffn)
    bm22, gm2, bm2_, gf2, bf2_ = (v.reshape(1, -1) for v in v2)
    bf12 = bf1.reshape(1, -1)
    bf22 = bf2.reshape(1, -1)
    return pl.pallas_call(
        _post_body,
        grid=(B, N // RN),
        in_specs=[
            pl.BlockSpec((1, RN, H), lambda b, i: (b, i, 0)),
            pl.BlockSpec((2, 1, RN, H // 2), lambda b, i: (0, b, i, 0)),
            pl.BlockSpec((1, 1, RN, H // 2), lambda b, i: (0, B, i, 0)),
            pl.BlockSpec(Wm2.shape, lambda b, i: (0, 0)),
            pl.BlockSpec((1, H), lambda b, i: (0, 0)),
            pl.BlockSpec((1, H), lambda b, i: (0, 0)),
            pl.BlockSpec((1, H), lambda b, i: (0, 0)),
            pl.BlockSpec(Wf1.shape, lambda b, i: (0, 0)),
            pl.BlockSpec((1, 2 * H), lambda b, i: (0, 0)),
            pl.BlockSpec(Wf2.shape, lambda b, i: (0, 0)),
            pl.BlockSpec((1, H), lambda b, i: (0, 0)),
            pl.BlockSpec((1, H), lambda b, i: (0, 0)),
            pl.BlockSpec((1, H), lambda b, i: (0, 0)),
        ],
        out_specs=pl.BlockSpec((1, RN, H), lambda b, i: (b, i, 0)),
        out_shape=jax.ShapeDtypeStruct((B, N, H), F32),
    )(h, agg4, agg4, Wm2, bm22, gm2, bm2_, Wf1, bf12, Wf2, bf22, gf2, bf2_)


def kernel(h, edge_attr, We, be, ge, bte, Wm1, bm1, Wm2, bm2, gmsg, bmsg,
           Wf1, bf1, Wf2, bf2, gffn, bffn, edge_index):
    B, N, H = h.shape
    E = edge_attr.shape[0]
    HALF = H // 2
    src = edge_index[0]
    dst = edge_index[1]

    hw2 = _hw_call(h.reshape(B * N, H), Wm1[:H])
    encw2 = _enc_call(edge_attr, We, be, ge, bte, Wm1[H:], bm1)

    agg4, NP = _sc_agg(hw2.reshape(2 * B * N, HALF),
                       encw2.reshape(2 * E, HALF),
                       src, dst, B, N, E, HALF)

    return _post_call(h, agg4, Wm2, bm2, gmsg, bmsg,
                      Wf1, bf1, Wf2, bf2, gffn, bffn)


# speedup vs baseline: 6.8699x; 1.3359x over previous
import functools

import jax
import jax.numpy as jnp
from jax import lax
from jax.experimental import pallas as pl
from jax.experimental.pallas import tpu as pltpu
from jax.experimental.pallas import tpu_sc as plsc

F32 = jnp.float32

_GC1 = 2.0 * 0.7978845608028654
_GC2 = _GC1 * 0.044715


def _gelu_exact(x):
    return 0.5 * x * (1.0 + lax.erf(x * 0.7071067811865476))


def _layer_norm(x, g, b, eps=1e-5):
    mu = jnp.mean(x, axis=-1, keepdims=True)
    var = jnp.mean((x - mu) ** 2, axis=-1, keepdims=True)
    return (x - mu) / jnp.sqrt(var + eps) * g + b


def _hw_body(h_ref, w_ref, out_ref):
    x = jnp.dot(h_ref[...], w_ref[...], preferred_element_type=F32)
    half = x.shape[-1] // 2
    out_ref[0] = x[:, :half]
    out_ref[1] = x[:, half:]


def _hw_call(h2d, w_top):
    BN, H = h2d.shape
    R = 1000
    return pl.pallas_call(
        _hw_body,
        grid=(BN // R,),
        in_specs=[
            pl.BlockSpec((R, H), lambda i: (i, 0)),
            pl.BlockSpec((H, H), lambda i: (0, 0)),
        ],
        out_specs=pl.BlockSpec((2, R, H // 2), lambda i: (0, i, 0)),
        out_shape=jax.ShapeDtypeStruct((2, BN, H // 2), F32),
    )(h2d, w_top)


def _enc_body(ea_ref, we_ref, be_ref, ge_ref, bte_ref, wb_ref, bm1_ref, out_ref):
    t = jnp.dot(ea_ref[...], we_ref[...], preferred_element_type=F32) + be_ref[...]
    t = _layer_norm(t, ge_ref[...], bte_ref[...])
    t = _gelu_exact(t)
    x = jnp.dot(t, wb_ref[...], preferred_element_type=F32) + bm1_ref[...]
    half = x.shape[-1] // 2
    out_ref[0] = x[:, :half]
    out_ref[1] = x[:, half:]


def _enc_call(edge_attr, We, be, ge, bte, w_bot, bm1):
    E, A = edge_attr.shape
    EH, H = w_bot.shape
    R = 1000
    be2, ge2, bte2, bm12 = (v.reshape(1, -1) for v in (be, ge, bte, bm1))
    return pl.pallas_call(
        _enc_body,
        grid=(E // R,),
        in_specs=[
            pl.BlockSpec((R, A), lambda i: (i, 0)),
            pl.BlockSpec((A, EH), lambda i: (0, 0)),
            pl.BlockSpec((1, EH), lambda i: (0, 0)),
            pl.BlockSpec((1, EH), lambda i: (0, 0)),
            pl.BlockSpec((1, EH), lambda i: (0, 0)),
            pl.BlockSpec((EH, H), lambda i: (0, 0)),
            pl.BlockSpec((1, H), lambda i: (0, 0)),
        ],
        out_specs=pl.BlockSpec((2, R, H // 2), lambda i: (0, i, 0)),
        out_shape=jax.ShapeDtypeStruct((2, E, H // 2), F32),
    )(edge_attr, We, be2, ge2, bte2, w_bot, bm12)


_NC, _NS = 2, 16
_K = 80
_ZR = 128


def _sc_agg(hw2, encw2, src, dst, B, N, E, HALF):
    EPT = E // _NS
    NCH = EPT // _K
    NP = -(-N // (_NS * _ZR)) * (_NS * _ZR)
    NPT = NP // _NS

    NTC = NPT // _K
    QC = NP // _K

    def body(hw_hbm, encw_hbm, src_hbm, dst_hbm, agg_hbm,
             sidx, didx, gidx, encb, rows, agg_sh, sem):
        c = lax.axis_index("c")
        s = lax.axis_index("s")

        zero16 = jnp.zeros((16,), F32)
        one16 = jnp.ones((16,), F32)

        for b in range(B + 1):
            def zrow_body(r, _):
                for l in range(HALF // 16):
                    rows[r, pl.ds(l * 16, 16)] = zero16
                return 0
            lax.fori_loop(0, _K, zrow_body, 0)
            for t in range(NTC):
                pltpu.sync_copy(rows, agg_sh.at[pl.ds(s * NPT + t * _K, _K)])
            plsc.subcore_barrier()

            if b < B:
                off = (c * B + b) * N

                def chunk_body(j, _):
                    base = s * EPT + j * _K
                    pltpu.sync_copy(src_hbm.at[pl.ds(base, _K)], sidx)
                    pltpu.sync_copy(dst_hbm.at[pl.ds(base, _K)], didx)
                    pltpu.sync_copy(encw_hbm.at[c * (E // _K) + s * NCH + j], encb)
                    cps = []
                    for v in range(_K // 16):
                        gvec = sidx[pl.ds(v * 16, 16)] + off
                        cps.append(pltpu.async_copy(
                            hw_hbm.at[gvec], rows.at[pl.ds(v * 16, 16)], sem))
                    for cp in cps:
                        cp.wait()

                    def row_body(r, _):
                        for l in range(HALF // 16):
                            x = rows[r, pl.ds(l * 16, 16)] + encb[r, pl.ds(l * 16, 16)]
                            xx = x * x
                            arg = x * (-_GC1 - _GC2 * xx)
                            rows[r, pl.ds(l * 16, 16)] = x / (1.0 + jnp.exp(arg))
                        return 0
                    lax.fori_loop(0, _K, row_body, 0)

                    for v in range(_K // 16):
                        dvec = didx[pl.ds(v * 16, 16)]
                        pltpu.sync_copy(rows.at[pl.ds(v * 16, 16)],
                                        agg_sh.at[dvec], add=True)
                    return 0
                lax.fori_loop(0, NCH, chunk_body, 0)
            else:
                def orow_body(r, _):
                    for l in range(HALF // 16):
                        rows[r, pl.ds(l * 16, 16)] = one16
                    return 0
                lax.fori_loop(0, _K, orow_body, 0)

                def deg_body(j, _):
                    base = s * EPT + j * _K
                    pltpu.sync_copy(dst_hbm.at[pl.ds(base, _K)], didx)
                    for v in range(_K // 16):
                        dvec = didx[pl.ds(v * 16, 16)]
                        pltpu.sync_copy(rows.at[pl.ds(v * 16, 16)],
                                        agg_sh.at[dvec], add=True)
                    return 0
                lax.fori_loop(0, NCH, deg_body, 0)
            plsc.subcore_barrier()

            for t in range(NTC):
                pltpu.sync_copy(agg_sh.at[pl.ds(s * NPT + t * _K, _K)], rows)
                pltpu.sync_copy(rows,
                                agg_hbm.at[(c * (B + 1) + b) * QC + s * NTC + t])

    mesh = plsc.VectorSubcoreMesh(core_axis_name="c", subcore_axis_name="s",
                                  num_cores=_NC, num_subcores=_NS)
    agg = pl.kernel(
        body,
        out_type=jax.ShapeDtypeStruct((_NC * (B + 1) * NP // _K, _K, HALF), F32),
        mesh=mesh,
        scratch_types=[
            pltpu.VMEM((_K,), jnp.int32),
            pltpu.VMEM((_K,), jnp.int32),
            pltpu.VMEM((_K,), jnp.int32),
            pltpu.VMEM((_K, HALF), F32),
            pltpu.VMEM((_K, HALF), F32),
            pltpu.VMEM_SHARED((NP, HALF), F32),
            pltpu.SemaphoreType.DMA,
        ],
    )(hw2, encw2.reshape(_NC * E // _K, _K, HALF), src, dst)
    return agg.reshape(_NC, B + 1, NP, HALF), NP


def _post_body(h_ref, agg_ref, deg_ref, wm2_ref, bm2_ref, gm_ref, bm_ref,
               wf1_ref, bf1_ref, wf2_ref, bf2_ref, gf_ref, bf_ref, out_ref):
    deg = deg_ref[0, 0][:, 0:1]
    inv = 1.0 / jnp.maximum(deg, 1.0)
    present = jnp.where(deg > 0.0, 1.0, 0.0)
    aggn = jnp.concatenate([agg_ref[0, 0], agg_ref[1, 0]], axis=-1) * inv
    aggregated = (jnp.dot(aggn, wm2_ref[...], preferred_element_type=F32)
                  + bm2_ref[...] * present)
    h1 = _layer_norm(h_ref[0] + aggregated, gm_ref[...], bm_ref[...])
    f = _gelu_exact(jnp.dot(h1, wf1_ref[...], preferred_element_type=F32)
                    + bf1_ref[...])
    ffn = jnp.dot(f, wf2_ref[...], preferred_element_type=F32) + bf2_ref[...]
    out_ref[0] = _layer_norm(h1 + ffn, gf_ref[...], bf_ref[...])


def _post_call(h, agg4, Wm2, bm2, gmsg, bmsg, Wf1, bf1, Wf2, bf2, gffn, bffn):
    B, N, H = h.shape
    RN = 1000
    v2 = (bm2, gmsg, bmsg, gffn, bffn)
    bm22, gm2, bm2_, gf2, bf2_ = (v.reshape(1, -1) for v in v2)
    bf12 = bf1.reshape(1, -1)
    bf22 = bf2.reshape(1, -1)
    return pl.pallas_call(
        _post_body,
        grid=(B, N // RN),
        in_specs=[
            pl.BlockSpec((1, RN, H), lambda b, i: (b, i, 0)),
            pl.BlockSpec((2, 1, RN, H // 2), lambda b, i: (0, b, i, 0)),
            pl.BlockSpec((1, 1, RN, H // 2), lambda b, i: (0, B, i, 0)),
            pl.BlockSpec(Wm2.shape, lambda b, i: (0, 0)),
            pl.BlockSpec((1, H), lambda b, i: (0, 0)),
            pl.BlockSpec((1, H), lambda b, i: (0, 0)),
            pl.BlockSpec((1, H), lambda b, i: (0, 0)),
            pl.BlockSpec(Wf1.shape, lambda b, i: (0, 0)),
            pl.BlockSpec((1, 2 * H), lambda b, i: (0, 0)),
            pl.BlockSpec(Wf2.shape, lambda b, i: (0, 0)),
            pl.BlockSpec((1, H), lambda b, i: (0, 0)),
            pl.BlockSpec((1, H), lambda b, i: (0, 0)),
            pl.BlockSpec((1, H), lambda b, i: (0, 0)),
        ],
        out_specs=pl.BlockSpec((1, RN, H), lambda b, i: (b, i, 0)),
        out_shape=jax.ShapeDtypeStruct((B, N, H), F32),
    )(h, agg4, agg4, Wm2, bm22, gm2, bm2_, Wf1, bf12, Wf2, bf22, gf2, bf2_)


def kernel(h, edge_attr, We, be, ge, bte, Wm1, bm1, Wm2, bm2, gmsg, bmsg,
           Wf1, bf1, Wf2, bf2, gffn, bffn, edge_index):
    B, N, H = h.shape
    E = edge_attr.shape[0]
    HALF = H // 2
    src = edge_index[0]
    dst = edge_index[1]

    hw2 = _hw_call(h.reshape(B * N, H), Wm1[:H])
    encw2 = _enc_call(edge_attr, We, be, ge, bte, Wm1[H:], bm1)

    agg4, NP = _sc_agg(hw2.reshape(2 * B * N, HALF),
                       encw2.reshape(2 * E, HALF),
                       src, dst, B, N, E, HALF)

    return _post_call(h, agg4, Wm2, bm2, gmsg, bmsg,
                      Wf1, bf1, Wf2, bf2, gffn, bffn)
